# baseline (speedup 1.0000x reference)
import functools
import numpy as np
import jax
import jax.numpy as jnp
from jax import lax
from jax.experimental import pallas as pl
from jax.experimental.pallas import tpu as pltpu
from jax.experimental.pallas import tpu_sc as plsc

B = 16384
NF = 26
TOTAL = 26 * 100000
PER_FIELD = 100000
D = 16
EPS = 1e-5
INV = np.float32(1.0 / np.sqrt(1.0 + EPS))
OFFSETS = np.arange(NF, dtype=np.int32) * PER_FIELD

NC, NS = 2, 16
NW = NC * NS
CH = 64
NCHUNK = B // CH
CPW = NCHUNK // NW
NIDX = NF * CH
NIDX16 = NIDX * D


def _fm_sc_body(idx16_hbm, idx_hbm, emb_hbm, lin_hbm, cross_hbm, linvals_hbm,
                idx16_v, idx_v, rows_v, linrows_v, cross_v, sem):
    wid = lax.axis_index("s") * NC + lax.axis_index("c")

    def chunk_body(c, carry0):
        gc = wid * CPW + c
        row_base = gc * CH
        pltpu.sync_copy(idx16_hbm.at[gc], idx16_v)
        pltpu.sync_copy(idx_hbm.at[gc], idx_v)
        cp1 = pltpu.async_copy(emb_hbm.at[idx16_v], rows_v, sem)
        cp2 = pltpu.async_copy(lin_hbm.at[idx_v], linrows_v, sem)
        cp1.wait()
        cp2.wait()

        def row_body(r, carry):
            base = r * (NF * D)
            s = jnp.zeros((16,), jnp.float32)
            sq = jnp.zeros((16,), jnp.float32)
            for f in range(NF):
                e = rows_v[pl.ds(base + f * D, 16)]
                s = s + e
                sq = sq + e * e
            cross_v[r] = 0.5 * (s * s - sq)
            return carry
        lax.fori_loop(0, CH, row_body, 0)

        pltpu.sync_copy(cross_v, cross_hbm.at[pl.ds(row_base, CH)])
        pltpu.sync_copy(linrows_v, linvals_hbm.at[pl.ds(row_base * NF, NIDX)])
        return carry0

    lax.fori_loop(0, CPW, chunk_body, 0)


_fm_sc = functools.partial(
    pl.kernel,
    out_type=[
        jax.ShapeDtypeStruct((B, D), jnp.float32),
        jax.ShapeDtypeStruct((B * NF,), jnp.float32),
    ],
    mesh=plsc.VectorSubcoreMesh(core_axis_name="c", subcore_axis_name="s"),
    scratch_types=[
        pltpu.VMEM((NIDX16,), jnp.int32),
        pltpu.VMEM((NIDX,), jnp.int32),
        pltpu.VMEM((NIDX16,), jnp.float32),
        pltpu.VMEM((NIDX,), jnp.float32),
        pltpu.VMEM((CH, D), jnp.float32),
        pltpu.SemaphoreType.DMA,
    ],
)(_fm_sc_body)


RB = 2048


def _mlp_body(cross_ref, linv_ref, lb_ref, g0, b0, W1, bb1, g1, bt1,
              W2, bb2, g2, bt2, W3, bb3, out_ref):
    lin = jnp.sum(linv_ref[...], axis=1)
    h = cross_ref[...] * (g0[...] * INV) + b0[...]
    z1 = lax.dot_general(h, W1[...], (((1,), (1,)), ((), ())),
                         preferred_element_type=jnp.float32)
    h1 = jnp.maximum((z1 + bb1[...]) * INV * g1[...] + bt1[...], 0.0)
    z2 = lax.dot_general(h1, W2[...], (((1,), (1,)), ((), ())),
                         preferred_element_type=jnp.float32)
    h2 = jnp.maximum((z2 + bb2[...]) * INV * g2[...] + bt2[...], 0.0)
    z3 = jnp.sum(h2 * W3[...], axis=1)
    out_ref[...] = jax.nn.sigmoid(lin + lb_ref[...] + z3 + bb3[...])


def _full(shape):
    return pl.BlockSpec(shape, lambda i: tuple(0 for _ in shape))


_mlp = pl.pallas_call(
    _mlp_body,
    grid=(B // RB,),
    in_specs=[
        pl.BlockSpec((RB, D), lambda i: (i, 0)),
        pl.BlockSpec((RB, NF), lambda i: (i, 0)),
        _full((1,)),
        _full((D,)), _full((D,)),
        _full((64, D)), _full((64,)), _full((64,)), _full((64,)),
        _full((32, 64)), _full((32,)), _full((32,)), _full((32,)),
        _full((1, 32)), _full((1,)),
    ],
    out_specs=pl.BlockSpec((RB,), lambda i: (i,)),
    out_shape=jax.ShapeDtypeStruct((B,), jnp.float32),
)


def kernel(users_feat, items_feat, emb_table, lin_table, lin_bias,
           g0, b0, W1, bb1, g1, bt1, W2, bb2, g2, bt2, W3, bb3):
    x = jnp.concatenate([users_feat, items_feat], axis=1) + jnp.asarray(
        OFFSETS, dtype=jnp.int32)
    xc = x.reshape(NCHUNK, CH * NF)
    x16 = (x * D)[:, :, None] + jnp.arange(D, dtype=jnp.int32)
    x16 = x16.reshape(NCHUNK, NIDX16)
    lin1 = lin_table.reshape(-1)
    cross, linvals = _fm_sc(x16, xc, emb_table.reshape(-1), lin1)
    return _mlp(cross, linvals.reshape(B, NF), lin_bias, g0, b0, W1, bb1,
                g1, bt1, W2, bb2, g2, bt2, W3, bb3)


# speedup vs baseline: 1.0004x; 1.0004x over previous
import functools
import numpy as np
import jax
import jax.numpy as jnp
from jax import lax
from jax.experimental import pallas as pl
from jax.experimental.pallas import tpu as pltpu
from jax.experimental.pallas import tpu_sc as plsc

B = 16384
NF = 26
TOTAL = 26 * 100000
PER_FIELD = 100000
D = 16
EPS = 1e-5
INV = np.float32(1.0 / np.sqrt(1.0 + EPS))
OFFSETS = np.arange(NF, dtype=np.int32) * PER_FIELD

NC, NS = 2, 16
NW = NC * NS
CH = 64
NCHUNK = B // CH
CPW = NCHUNK // NW
NIDX = NF * CH
NIDX16 = NIDX * D


def _fm_sc_body(idx16_hbm, idx_hbm, emb_hbm, lin_hbm, cross_hbm, linvals_hbm,
                idx16_v, idx_v, rows_v, linrows_v, cross_v, sem):
    wid = lax.axis_index("s") * NC + lax.axis_index("c")

    def chunk_body(c, carry0):
        gc = wid * CPW + c
        row_base = gc * CH
        pltpu.sync_copy(idx16_hbm.at[gc], idx16_v)
        pltpu.sync_copy(idx_hbm.at[gc], idx_v)
        cp1 = pltpu.async_copy(emb_hbm.at[idx16_v], rows_v, sem)
        cp2 = pltpu.async_copy(lin_hbm.at[idx_v], linrows_v, sem)
        cp1.wait()
        cp2.wait()

        def row_body(r, carry):
            base = r * (NF * D)
            s = jnp.zeros((16,), jnp.float32)
            sq = jnp.zeros((16,), jnp.float32)
            for f in range(NF):
                e = rows_v[pl.ds(base + f * D, 16)]
                s = s + e
                sq = sq + e * e
            cross_v[r] = 0.5 * (s * s - sq)
            return carry
        lax.fori_loop(0, CH, row_body, 0)

        pltpu.sync_copy(cross_v, cross_hbm.at[pl.ds(row_base, CH)])
        pltpu.sync_copy(linrows_v, linvals_hbm.at[pl.ds(row_base * NF, NIDX)])
        return carry0

    lax.fori_loop(0, CPW, chunk_body, 0)


_fm_sc = functools.partial(
    pl.kernel,
    out_type=[
        jax.ShapeDtypeStruct((B, D), jnp.float32),
        jax.ShapeDtypeStruct((B * NF,), jnp.float32),
    ],
    mesh=plsc.VectorSubcoreMesh(core_axis_name="c", subcore_axis_name="s"),
    scratch_types=[
        pltpu.VMEM((NIDX16,), jnp.int32),
        pltpu.VMEM((NIDX,), jnp.int32),
        pltpu.VMEM((NIDX16,), jnp.float32),
        pltpu.VMEM((NIDX,), jnp.float32),
        pltpu.VMEM((CH, D), jnp.float32),
        pltpu.SemaphoreType.DMA,
    ],
)(_fm_sc_body)


RB = 2048


def _mlp_body(cross_ref, linv_ref, lb_ref, g0, b0, W1, bb1, g1, bt1,
              W2, bb2, g2, bt2, W3, bb3, out_ref):
    lin = jnp.sum(linv_ref[...], axis=1)
    h = cross_ref[...] * (g0[...] * INV) + b0[...]
    z1 = lax.dot_general(h, W1[...], (((1,), (1,)), ((), ())),
                         preferred_element_type=jnp.float32)
    h1 = jnp.maximum((z1 + bb1[...]) * INV * g1[...] + bt1[...], 0.0)
    z2 = lax.dot_general(h1, W2[...], (((1,), (1,)), ((), ())),
                         preferred_element_type=jnp.float32)
    h2 = jnp.maximum((z2 + bb2[...]) * INV * g2[...] + bt2[...], 0.0)
    z3 = jnp.sum(h2 * W3[...], axis=1)
    out_ref[...] = jax.nn.sigmoid(lin + lb_ref[...] + z3 + bb3[...])


def _full(shape):
    return pl.BlockSpec(shape, lambda i: tuple(0 for _ in shape))


_mlp = pl.pallas_call(
    _mlp_body,
    grid=(B // RB,),
    in_specs=[
        pl.BlockSpec((RB, D), lambda i: (i, 0)),
        pl.BlockSpec((RB, NF), lambda i: (i, 0)),
        _full((1,)),
        _full((D,)), _full((D,)),
        _full((64, D)), _full((64,)), _full((64,)), _full((64,)),
        _full((32, 64)), _full((32,)), _full((32,)), _full((32,)),
        _full((1, 32)), _full((1,)),
    ],
    out_specs=pl.BlockSpec((RB,), lambda i: (i,)),
    out_shape=jax.ShapeDtypeStruct((B,), jnp.float32),
)


def kernel(users_feat, items_feat, emb_table, lin_table, lin_bias,
           g0, b0, W1, bb1, g1, bt1, W2, bb2, g2, bt2, W3, bb3):
    x = jnp.concatenate([users_feat, items_feat], axis=1) + jnp.asarray(
        OFFSETS, dtype=jnp.int32)
    xc = x.reshape(NCHUNK, CH * NF)
    x16 = (x * D)[:, :, None] + jnp.arange(D, dtype=jnp.int32)
    x16 = x16.reshape(NCHUNK, NIDX16)
    lin1 = lin_table.reshape(-1)
    emb_ref = jax.new_ref(emb_table.reshape(-1))
    cross, linvals = _fm_sc(x16, xc, emb_ref, lin1)
    return _mlp(cross, linvals.reshape(B, NF), lin_bias, g0, b0, W1, bb1,
                g1, bt1, W2, bb2, g2, bt2, W3, bb3)


# speedup vs baseline: 1.3843x; 1.3837x over previous
import functools
import numpy as np
import jax
import jax.numpy as jnp
from jax import lax
from jax.experimental import pallas as pl
from jax.experimental.pallas import tpu as pltpu
from jax.experimental.pallas import tpu_sc as plsc

B = 16384
NF = 26
TOTAL = 26 * 100000
PER_FIELD = 100000
D = 16
EPS = 1e-5
INV = np.float32(1.0 / np.sqrt(1.0 + EPS))
OFFSETS = np.arange(NF, dtype=np.int32) * PER_FIELD

NC, NS = 2, 16
NW = NC * NS
CH = 128
NCHUNK = B // CH
CPW = NCHUNK // NW
NIDX = NF * CH
G16 = CH // 16


def _fm_sc_body(idx_hbm, emb_hbm, lin_hbm, cross_hbm, linsum_hbm,
                idx_v, rows_v, linrows_v, cross_v, lin_v, sem):
    wid = lax.axis_index("s") * NC + lax.axis_index("c")

    def chunk_body(c, carry0):
        gc = wid * CPW + c
        row_base = gc * CH
        pltpu.sync_copy(idx_hbm.at[gc], idx_v)
        cp1 = pltpu.async_copy(emb_hbm.at[idx_v], rows_v, sem)
        cp2 = pltpu.async_copy(lin_hbm.at[idx_v], linrows_v, sem)
        cp1.wait()
        cp2.wait()

        def row_body(r, carry):
            s = jnp.zeros((16,), jnp.float32)
            sq = jnp.zeros((16,), jnp.float32)
            for f in range(NF):
                e = rows_v[f * CH + r]
                s = s + e
                sq = sq + e * e
            cross_v[r] = 0.5 * (s * s - sq)
            return carry
        lax.fori_loop(0, CH, row_body, 0)

        for g in range(G16):
            acc = jnp.zeros((16,), jnp.float32)
            for f in range(NF):
                acc = acc + linrows_v[pl.ds(f * CH + g * 16, 16)]
            lin_v[pl.ds(g * 16, 16)] = acc

        pltpu.sync_copy(cross_v, cross_hbm.at[pl.ds(row_base, CH)])
        pltpu.sync_copy(lin_v, linsum_hbm.at[pl.ds(row_base, CH)])
        return carry0

    lax.fori_loop(0, CPW, chunk_body, 0)


_fm_sc = functools.partial(
    pl.kernel,
    out_type=[
        jax.ShapeDtypeStruct((B, D), jnp.float32),
        jax.ShapeDtypeStruct((B,), jnp.float32),
    ],
    mesh=plsc.VectorSubcoreMesh(core_axis_name="c", subcore_axis_name="s"),
    compiler_params=pltpu.CompilerParams(use_tc_tiling_on_sc=False),
    scratch_types=[
        pltpu.VMEM((NIDX,), jnp.int32),
        pltpu.VMEM((NIDX, D), jnp.float32),
        pltpu.VMEM((NIDX,), jnp.float32),
        pltpu.VMEM((CH, D), jnp.float32),
        pltpu.VMEM((CH,), jnp.float32),
        pltpu.SemaphoreType.DMA,
    ],
)(_fm_sc_body)


RB = 2048


def _mlp_body(cross_ref, lin_ref, lb_ref, g0, b0, W1, bb1, g1, bt1,
              W2, bb2, g2, bt2, W3, bb3, out_ref):
    lin = lin_ref[...]
    h = cross_ref[...] * (g0[...] * INV) + b0[...]
    z1 = lax.dot_general(h, W1[...], (((1,), (1,)), ((), ())),
                         preferred_element_type=jnp.float32)
    h1 = jnp.maximum((z1 + bb1[...]) * INV * g1[...] + bt1[...], 0.0)
    z2 = lax.dot_general(h1, W2[...], (((1,), (1,)), ((), ())),
                         preferred_element_type=jnp.float32)
    h2 = jnp.maximum((z2 + bb2[...]) * INV * g2[...] + bt2[...], 0.0)
    z3 = jnp.sum(h2 * W3[...], axis=1)
    out_ref[...] = jax.nn.sigmoid(lin + lb_ref[...] + z3 + bb3[...])


def _full(shape):
    return pl.BlockSpec(shape, lambda i: tuple(0 for _ in shape))


_mlp = pl.pallas_call(
    _mlp_body,
    grid=(B // RB,),
    in_specs=[
        pl.BlockSpec((RB, D), lambda i: (i, 0)),
        pl.BlockSpec((RB,), lambda i: (i,)),
        _full((1,)),
        _full((D,)), _full((D,)),
        _full((64, D)), _full((64,)), _full((64,)), _full((64,)),
        _full((32, 64)), _full((32,)), _full((32,)), _full((32,)),
        _full((1, 32)), _full((1,)),
    ],
    out_specs=pl.BlockSpec((RB,), lambda i: (i,)),
    out_shape=jax.ShapeDtypeStruct((B,), jnp.float32),
)


def kernel(users_feat, items_feat, emb_table, lin_table, lin_bias,
           g0, b0, W1, bb1, g1, bt1, W2, bb2, g2, bt2, W3, bb3):
    x = jnp.concatenate([users_feat, items_feat], axis=1) + jnp.asarray(
        OFFSETS, dtype=jnp.int32)
    xc = x.reshape(NCHUNK, CH, NF).transpose(0, 2, 1).reshape(NCHUNK, NIDX)
    lin1 = lin_table.reshape(-1)
    cross, linsum = _fm_sc(xc, emb_table, lin1)
    return _mlp(cross, linsum, lin_bias, g0, b0, W1, bb1,
                g1, bt1, W2, bb2, g2, bt2, W3, bb3)


# speedup vs baseline: 1.3904x; 1.0044x over previous
import functools
import numpy as np
import jax
import jax.numpy as jnp
from jax import lax
from jax.experimental import pallas as pl
from jax.experimental.pallas import tpu as pltpu
from jax.experimental.pallas import tpu_sc as plsc

B = 16384
NF = 26
TOTAL = 26 * 100000
PER_FIELD = 100000
D = 16
EPS = 1e-5
INV = np.float32(1.0 / np.sqrt(1.0 + EPS))
OFFSETS = np.arange(NF, dtype=np.int32) * PER_FIELD

NC, NS = 2, 16
NW = NC * NS
CH = 64
NCHUNK = B // CH
CPW = NCHUNK // NW
NIDX = NF * CH
G16 = CH // 16
RPW = B // NW


def _fm_sc_body(idx_hbm, emb_hbm, lin_hbm, cross_hbm, linsum_hbm,
                idx_all, rows0, rows1, lin0, lin1, cross_v, lin_v,
                sem0, sem1):
    wid = lax.axis_index("s") * NC + lax.axis_index("c")
    base_chunk = wid * CPW
    pltpu.sync_copy(idx_hbm.at[wid], idx_all)

    rows_bufs = (rows0, rows1)
    lin_bufs = (lin0, lin1)
    sems = (sem0, sem1)

    def issue(c):
        p = c % 2
        isl = idx_all.at[pl.ds(c * NIDX, NIDX)]
        cp1 = pltpu.async_copy(emb_hbm.at[isl], rows_bufs[p], sems[p])
        cp2 = pltpu.async_copy(lin_hbm.at[isl], lin_bufs[p], sems[p])
        return cp1, cp2

    pend = issue(0)
    for c in range(CPW):
        p = c % 2
        nxt = issue(c + 1) if c + 1 < CPW else None
        pend[0].wait()
        pend[1].wait()
        rows_v, linrows_v = rows_bufs[p], lin_bufs[p]

        def row_body(r, carry):
            s = jnp.zeros((16,), jnp.float32)
            sq = jnp.zeros((16,), jnp.float32)
            for f in range(NF):
                e = rows_v[f * CH + r]
                s = s + e
                sq = sq + e * e
            cross_v[r] = 0.5 * (s * s - sq)
            return carry
        lax.fori_loop(0, CH, row_body, 0)

        for g in range(G16):
            acc = jnp.zeros((16,), jnp.float32)
            for f in range(NF):
                acc = acc + linrows_v[pl.ds(f * CH + g * 16, 16)]
            lin_v[pl.ds(g * 16, 16)] = acc

        row_base = (base_chunk + c) * CH
        pltpu.sync_copy(cross_v, cross_hbm.at[pl.ds(row_base, CH)])
        pltpu.sync_copy(lin_v, linsum_hbm.at[pl.ds(row_base, CH)])
        pend = nxt


_fm_sc = functools.partial(
    pl.kernel,
    out_type=[
        jax.ShapeDtypeStruct((B, D), jnp.float32),
        jax.ShapeDtypeStruct((B,), jnp.float32),
    ],
    mesh=plsc.VectorSubcoreMesh(core_axis_name="c", subcore_axis_name="s"),
    compiler_params=pltpu.CompilerParams(use_tc_tiling_on_sc=False),
    scratch_types=[
        pltpu.VMEM((RPW * NF,), jnp.int32),
        pltpu.VMEM((NIDX, D), jnp.float32),
        pltpu.VMEM((NIDX, D), jnp.float32),
        pltpu.VMEM((NIDX,), jnp.float32),
        pltpu.VMEM((NIDX,), jnp.float32),
        pltpu.VMEM((CH, D), jnp.float32),
        pltpu.VMEM((CH,), jnp.float32),
        pltpu.SemaphoreType.DMA,
        pltpu.SemaphoreType.DMA,
    ],
)(_fm_sc_body)


RB = 2048


def _mlp_body(cross_ref, lin_ref, lb_ref, g0, b0, W1, bb1, g1, bt1,
              W2, bb2, g2, bt2, W3, bb3, out_ref):
    lin = lin_ref[...]
    h = cross_ref[...] * (g0[...] * INV) + b0[...]
    z1 = lax.dot_general(h, W1[...], (((1,), (1,)), ((), ())),
                         preferred_element_type=jnp.float32)
    h1 = jnp.maximum((z1 + bb1[...]) * INV * g1[...] + bt1[...], 0.0)
    z2 = lax.dot_general(h1, W2[...], (((1,), (1,)), ((), ())),
                         preferred_element_type=jnp.float32)
    h2 = jnp.maximum((z2 + bb2[...]) * INV * g2[...] + bt2[...], 0.0)
    z3 = jnp.sum(h2 * W3[...], axis=1)
    out_ref[...] = jax.nn.sigmoid(lin + lb_ref[...] + z3 + bb3[...])


def _full(shape):
    return pl.BlockSpec(shape, lambda i: tuple(0 for _ in shape))


_mlp = pl.pallas_call(
    _mlp_body,
    grid=(B // RB,),
    in_specs=[
        pl.BlockSpec((RB, D), lambda i: (i, 0)),
        pl.BlockSpec((RB,), lambda i: (i,)),
        _full((1,)),
        _full((D,)), _full((D,)),
        _full((64, D)), _full((64,)), _full((64,)), _full((64,)),
        _full((32, 64)), _full((32,)), _full((32,)), _full((32,)),
        _full((1, 32)), _full((1,)),
    ],
    out_specs=pl.BlockSpec((RB,), lambda i: (i,)),
    out_shape=jax.ShapeDtypeStruct((B,), jnp.float32),
)


def kernel(users_feat, items_feat, emb_table, lin_table, lin_bias,
           g0, b0, W1, bb1, g1, bt1, W2, bb2, g2, bt2, W3, bb3):
    x = jnp.concatenate([users_feat, items_feat], axis=1) + jnp.asarray(
        OFFSETS, dtype=jnp.int32)
    xc = x.reshape(NW, CPW, CH, NF).transpose(0, 1, 3, 2).reshape(NW, CPW * NIDX)
    lin1 = lin_table.reshape(-1)
    cross, linsum = _fm_sc(xc, emb_table, lin1)
    return _mlp(cross, linsum, lin_bias, g0, b0, W1, bb1,
                g1, bt1, W2, bb2, g2, bt2, W3, bb3)
